# baseline (device time: 80281 ns/iter reference)
import jax
import jax.numpy as jnp
from jax import lax
from jax.experimental import pallas as pl
from jax.experimental.pallas import tpu as pltpu

NC = 4


def kernel(x):
    m, n = x.shape
    h = m // 2
    q = m // 4
    e = m // 8
    ec = e // NC

    def body(x_hbm, out_hbm, xf_a, xf_b, xo_a, xo_b, fwd_a, fwd_b,
             rv_a1, rv_b1, rv_a2, rv_b2, st_a, st_b,
             local_sems, send_sems, recv_sems):
        my_x = lax.axis_index("x")
        my_y = lax.axis_index("y")
        y_nbr = (my_x, 1 - my_y)
        x_nbr = (1 - my_x, my_y)

        sem_ctr = [0]

        def exch(src, dst, nbr):
            i = sem_ctr[0]
            sem_ctr[0] += 1
            return pltpu.make_async_remote_copy(
                src_ref=src, dst_ref=dst,
                send_sem=send_sems.at[i], recv_sem=recv_sems.at[i],
                device_id=nbr, device_id_type=pl.DeviceIdType.MESH,
            )

        sa = (1 - my_y) * q
        oa = my_y * q
        sb = h + (1 - my_x) * q
        ob = h + my_x * q
        fa = (1 - my_x) * e
        na = my_x * e
        fb = (1 - my_y) * e
        nb = my_y * e
        a_blk = oa + na
        b_blk = ob + nb

        def ds(base, c):
            return pl.ds(base + c * ec, ec)

        stage = []
        for i, (src_row, dst) in enumerate((
                (oa + fa, xf_a), (ob + fb, xf_b),
                (oa + na, xo_a), (ob + nb, xo_b))):
            cp = pltpu.make_async_copy(
                x_hbm.at[pl.ds(src_row, e)], dst, local_sems.at[i])
            cp.start()
            stage.append(cp)

        barrier = pltpu.get_barrier_semaphore()
        for nbr in (y_nbr, x_nbr):
            pl.semaphore_signal(
                barrier, inc=1, device_id=nbr,
                device_id_type=pl.DeviceIdType.MESH,
            )
        pl.semaphore_wait(barrier, 2)

        a1, b1 = {}, {}
        for p, arow in ((0, fa), (1, na)):
            for c in range(NC):
                a1[p, c] = exch(x_hbm.at[ds(sa + arow, c)],
                                rv_a1.at[ds(arow, c)], y_nbr)
        for p, brow in ((0, fb), (1, nb)):
            for c in range(NC):
                b1[p, c] = exch(x_hbm.at[ds(sb + brow, c)],
                                rv_b1.at[ds(brow, c)], x_nbr)
        for c in range(NC):
            a1[0, c].start()
            b1[0, c].start()
        for c in range(NC):
            a1[1, c].start()
            b1[1, c].start()

        stage[0].wait()
        stage[1].wait()

        a2, b2 = {}, {}
        for c in range(NC):
            a1[0, c].wait_recv()
            fwd_a[ds(0, c), :] = xf_a[ds(0, c), :] + rv_a1[ds(fa, c), :]
            a2[c] = exch(fwd_a.at[ds(0, c)], rv_a2.at[ds(0, c)], x_nbr)
            a2[c].start()

            b1[0, c].wait_recv()
            fwd_b[ds(0, c), :] = xf_b[ds(0, c), :] + rv_b1[ds(fb, c), :]
            b2[c] = exch(fwd_b.at[ds(0, c)], rv_b2.at[ds(0, c)], y_nbr)
            b2[c].start()

        stage[2].wait()
        stage[3].wait()

        a3, b3, a4p1, b4p1 = {}, {}, {}, {}
        out_cp = []
        for c in range(NC):
            a1[1, c].wait_recv()
            a2[c].wait_recv()
            st_a[ds(0, c), :] = (
                xo_a[ds(0, c), :] + rv_a1[ds(na, c), :] + rv_a2[ds(0, c), :])
            cp = pltpu.make_async_copy(
                st_a.at[ds(0, c)], out_hbm.at[ds(a_blk, c)],
                local_sems.at[4 + c])
            cp.start()
            out_cp.append(cp)
            a4p1[c] = exch(st_a.at[ds(0, c)],
                           out_hbm.at[ds(a_blk, c)], y_nbr)
            a4p1[c].start()
            a3[c] = exch(st_a.at[ds(0, c)],
                         out_hbm.at[ds(a_blk, c)], x_nbr)
            a3[c].start()

            b1[1, c].wait_recv()
            b2[c].wait_recv()
            st_b[ds(0, c), :] = (
                xo_b[ds(0, c), :] + rv_b1[ds(nb, c), :] + rv_b2[ds(0, c), :])
            cp = pltpu.make_async_copy(
                st_b.at[ds(0, c)], out_hbm.at[ds(b_blk, c)],
                local_sems.at[4 + NC + c])
            cp.start()
            out_cp.append(cp)
            b4p1[c] = exch(st_b.at[ds(0, c)],
                           out_hbm.at[ds(b_blk, c)], x_nbr)
            b4p1[c].start()
            b3[c] = exch(st_b.at[ds(0, c)],
                         out_hbm.at[ds(b_blk, c)], y_nbr)
            b3[c].start()

        a4p2, b4p2 = {}, {}
        for c in range(NC):
            a3[c].wait_recv()
            a4p2[c] = exch(out_hbm.at[ds(oa + fa, c)],
                           out_hbm.at[ds(oa + fa, c)], y_nbr)
            a4p2[c].start()
            b3[c].wait_recv()
            b4p2[c] = exch(out_hbm.at[ds(ob + fb, c)],
                           out_hbm.at[ds(ob + fb, c)], x_nbr)
            b4p2[c].start()

        for c in range(NC):
            a4p1[c].wait_recv()
            b4p1[c].wait_recv()
            a4p2[c].wait_recv()
            b4p2[c].wait_recv()
        for cp in out_cp:
            cp.wait()

        for grp in (a1, b1, a2, b2, a3, b3, a4p1, b4p1, a4p2, b4p2):
            for r in grp.values():
                r.wait_send()

    n_sems = 12 * NC
    return pl.pallas_call(
        body,
        out_shape=jax.ShapeDtypeStruct((m, n), jnp.float32),
        in_specs=[pl.BlockSpec(memory_space=pl.ANY)],
        out_specs=pl.BlockSpec(memory_space=pl.ANY),
        scratch_shapes=[
            pltpu.VMEM((e, n), jnp.float32),
            pltpu.VMEM((e, n), jnp.float32),
            pltpu.VMEM((e, n), jnp.float32),
            pltpu.VMEM((e, n), jnp.float32),
            pltpu.VMEM((e, n), jnp.float32),
            pltpu.VMEM((e, n), jnp.float32),
            pltpu.VMEM((q, n), jnp.float32),
            pltpu.VMEM((q, n), jnp.float32),
            pltpu.VMEM((e, n), jnp.float32),
            pltpu.VMEM((e, n), jnp.float32),
            pltpu.VMEM((e, n), jnp.float32),
            pltpu.VMEM((e, n), jnp.float32),
            pltpu.SemaphoreType.DMA((4 + 2 * NC,)),
            pltpu.SemaphoreType.DMA((n_sems,)),
            pltpu.SemaphoreType.DMA((n_sems,)),
        ],
        compiler_params=pltpu.CompilerParams(collective_id=0),
    )(x)


# device time: 79803 ns/iter; 1.0060x vs baseline; 1.0060x over previous
import jax
import jax.numpy as jnp
from jax import lax
from jax.experimental import pallas as pl
from jax.experimental.pallas import tpu as pltpu

NC = 2


def kernel(x):
    m, n = x.shape
    h = m // 2
    q = m // 4
    e = m // 8
    ec = e // NC

    def body(x_hbm, out_hbm, xf_a, xf_b, xo_a, xo_b, fwd_a, fwd_b,
             rv_a1, rv_b1, rv_a2, rv_b2, st_a, st_b,
             local_sems, send_sems, recv_sems):
        my_x = lax.axis_index("x")
        my_y = lax.axis_index("y")
        y_nbr = (my_x, 1 - my_y)
        x_nbr = (1 - my_x, my_y)

        sem_ctr = [0]

        def exch(src, dst, nbr):
            i = sem_ctr[0]
            sem_ctr[0] += 1
            return pltpu.make_async_remote_copy(
                src_ref=src, dst_ref=dst,
                send_sem=send_sems.at[i], recv_sem=recv_sems.at[i],
                device_id=nbr, device_id_type=pl.DeviceIdType.MESH,
            )

        sa = (1 - my_y) * q
        oa = my_y * q
        sb = h + (1 - my_x) * q
        ob = h + my_x * q
        fa = (1 - my_x) * e
        na = my_x * e
        fb = (1 - my_y) * e
        nb = my_y * e
        a_blk = oa + na
        b_blk = ob + nb

        def ds(base, c):
            return pl.ds(base + c * ec, ec)

        stage = []
        for i, (src_row, dst) in enumerate((
                (oa + fa, xf_a), (ob + fb, xf_b),
                (oa + na, xo_a), (ob + nb, xo_b))):
            cp = pltpu.make_async_copy(
                x_hbm.at[pl.ds(src_row, e)], dst, local_sems.at[i])
            cp.start()
            stage.append(cp)

        barrier = pltpu.get_barrier_semaphore()
        for nbr in (y_nbr, x_nbr):
            pl.semaphore_signal(
                barrier, inc=1, device_id=nbr,
                device_id_type=pl.DeviceIdType.MESH,
            )
        pl.semaphore_wait(barrier, 2)

        a1, b1 = {}, {}
        for p, arow in ((0, fa), (1, na)):
            for c in range(NC):
                a1[p, c] = exch(x_hbm.at[ds(sa + arow, c)],
                                rv_a1.at[ds(arow, c)], y_nbr)
        for p, brow in ((0, fb), (1, nb)):
            for c in range(NC):
                b1[p, c] = exch(x_hbm.at[ds(sb + brow, c)],
                                rv_b1.at[ds(brow, c)], x_nbr)
        for c in range(NC):
            a1[0, c].start()
            b1[0, c].start()
        for c in range(NC):
            a1[1, c].start()
            b1[1, c].start()

        stage[0].wait()
        stage[1].wait()

        a2, b2 = {}, {}
        for c in range(NC):
            a1[0, c].wait_recv()
            fwd_a[ds(0, c), :] = xf_a[ds(0, c), :] + rv_a1[ds(fa, c), :]
            a2[c] = exch(fwd_a.at[ds(0, c)], rv_a2.at[ds(0, c)], x_nbr)
            a2[c].start()

            b1[0, c].wait_recv()
            fwd_b[ds(0, c), :] = xf_b[ds(0, c), :] + rv_b1[ds(fb, c), :]
            b2[c] = exch(fwd_b.at[ds(0, c)], rv_b2.at[ds(0, c)], y_nbr)
            b2[c].start()

        stage[2].wait()
        stage[3].wait()

        a3, b3, a4p1, b4p1 = {}, {}, {}, {}
        out_cp = []
        for c in range(NC):
            a1[1, c].wait_recv()
            a2[c].wait_recv()
            st_a[ds(0, c), :] = (
                xo_a[ds(0, c), :] + rv_a1[ds(na, c), :] + rv_a2[ds(0, c), :])
            cp = pltpu.make_async_copy(
                st_a.at[ds(0, c)], out_hbm.at[ds(a_blk, c)],
                local_sems.at[4 + c])
            cp.start()
            out_cp.append(cp)
            a4p1[c] = exch(st_a.at[ds(0, c)],
                           out_hbm.at[ds(a_blk, c)], y_nbr)
            a4p1[c].start()
            a3[c] = exch(st_a.at[ds(0, c)],
                         out_hbm.at[ds(a_blk, c)], x_nbr)
            a3[c].start()

            b1[1, c].wait_recv()
            b2[c].wait_recv()
            st_b[ds(0, c), :] = (
                xo_b[ds(0, c), :] + rv_b1[ds(nb, c), :] + rv_b2[ds(0, c), :])
            cp = pltpu.make_async_copy(
                st_b.at[ds(0, c)], out_hbm.at[ds(b_blk, c)],
                local_sems.at[4 + NC + c])
            cp.start()
            out_cp.append(cp)
            b4p1[c] = exch(st_b.at[ds(0, c)],
                           out_hbm.at[ds(b_blk, c)], x_nbr)
            b4p1[c].start()
            b3[c] = exch(st_b.at[ds(0, c)],
                         out_hbm.at[ds(b_blk, c)], y_nbr)
            b3[c].start()

        a4p2, b4p2 = {}, {}
        for c in range(NC):
            a3[c].wait_recv()
            a4p2[c] = exch(out_hbm.at[ds(oa + fa, c)],
                           out_hbm.at[ds(oa + fa, c)], y_nbr)
            a4p2[c].start()
            b3[c].wait_recv()
            b4p2[c] = exch(out_hbm.at[ds(ob + fb, c)],
                           out_hbm.at[ds(ob + fb, c)], x_nbr)
            b4p2[c].start()

        for c in range(NC):
            a4p1[c].wait_recv()
            b4p1[c].wait_recv()
            a4p2[c].wait_recv()
            b4p2[c].wait_recv()
        for cp in out_cp:
            cp.wait()

        for grp in (a1, b1, a2, b2, a3, b3, a4p1, b4p1, a4p2, b4p2):
            for r in grp.values():
                r.wait_send()

    n_sems = 12 * NC
    return pl.pallas_call(
        body,
        out_shape=jax.ShapeDtypeStruct((m, n), jnp.float32),
        in_specs=[pl.BlockSpec(memory_space=pl.ANY)],
        out_specs=pl.BlockSpec(memory_space=pl.ANY),
        scratch_shapes=[
            pltpu.VMEM((e, n), jnp.float32),
            pltpu.VMEM((e, n), jnp.float32),
            pltpu.VMEM((e, n), jnp.float32),
            pltpu.VMEM((e, n), jnp.float32),
            pltpu.VMEM((e, n), jnp.float32),
            pltpu.VMEM((e, n), jnp.float32),
            pltpu.VMEM((q, n), jnp.float32),
            pltpu.VMEM((q, n), jnp.float32),
            pltpu.VMEM((e, n), jnp.float32),
            pltpu.VMEM((e, n), jnp.float32),
            pltpu.VMEM((e, n), jnp.float32),
            pltpu.VMEM((e, n), jnp.float32),
            pltpu.SemaphoreType.DMA((4 + 2 * NC,)),
            pltpu.SemaphoreType.DMA((n_sems,)),
            pltpu.SemaphoreType.DMA((n_sems,)),
        ],
        input_output_aliases={0: 0},
        compiler_params=pltpu.CompilerParams(collective_id=0),
    )(x)


# device time: 79397 ns/iter; 1.0111x vs baseline; 1.0051x over previous
import jax
import jax.numpy as jnp
from jax import lax
from jax.experimental import pallas as pl
from jax.experimental.pallas import tpu as pltpu

NC = 2


def kernel(x):
    m, n = x.shape
    h = m // 2
    q = m // 4
    e = m // 8
    ec = e // NC

    def body(x_ref, out_ref, fwd_a, fwd_b, rv_a1, rv_b1, rv_a2, rv_b2,
             send_sems, recv_sems):
        my_x = lax.axis_index("x")
        my_y = lax.axis_index("y")
        y_nbr = (my_x, 1 - my_y)
        x_nbr = (1 - my_x, my_y)

        sem_ctr = [0]

        def exch(src, dst, nbr):
            i = sem_ctr[0]
            sem_ctr[0] += 1
            return pltpu.make_async_remote_copy(
                src_ref=src, dst_ref=dst,
                send_sem=send_sems.at[i], recv_sem=recv_sems.at[i],
                device_id=nbr, device_id_type=pl.DeviceIdType.MESH,
            )

        barrier = pltpu.get_barrier_semaphore()
        for nbr in (y_nbr, x_nbr):
            pl.semaphore_signal(
                barrier, inc=1, device_id=nbr,
                device_id_type=pl.DeviceIdType.MESH,
            )
        pl.semaphore_wait(barrier, 2)

        sa = (1 - my_y) * q
        oa = my_y * q
        sb = h + (1 - my_x) * q
        ob = h + my_x * q
        fa = (1 - my_x) * e
        na = my_x * e
        fb = (1 - my_y) * e
        nb = my_y * e
        a_blk = oa + na
        b_blk = ob + nb

        def ds(base, c):
            return pl.ds(base + c * ec, ec)

        a1, b1 = {}, {}
        for p, arow in ((0, fa), (1, na)):
            for c in range(NC):
                a1[p, c] = exch(x_ref.at[ds(sa + arow, c)],
                                rv_a1.at[ds(arow, c)], y_nbr)
        for p, brow in ((0, fb), (1, nb)):
            for c in range(NC):
                b1[p, c] = exch(x_ref.at[ds(sb + brow, c)],
                                rv_b1.at[ds(brow, c)], x_nbr)
        for c in range(NC):
            a1[0, c].start()
            b1[0, c].start()
        for c in range(NC):
            a1[1, c].start()
            b1[1, c].start()

        a2, b2 = {}, {}
        for c in range(NC):
            a1[0, c].wait_recv()
            fwd_a[ds(0, c), :] = (
                x_ref[ds(oa + fa, c), :] + rv_a1[ds(fa, c), :])
            a2[c] = exch(fwd_a.at[ds(0, c)], rv_a2.at[ds(0, c)], x_nbr)
            a2[c].start()

            b1[0, c].wait_recv()
            fwd_b[ds(0, c), :] = (
                x_ref[ds(ob + fb, c), :] + rv_b1[ds(fb, c), :])
            b2[c] = exch(fwd_b.at[ds(0, c)], rv_b2.at[ds(0, c)], y_nbr)
            b2[c].start()

        a3, b3, a4p1, b4p1 = {}, {}, {}, {}
        for c in range(NC):
            a1[1, c].wait_recv()
            a2[c].wait_recv()
            out_ref[ds(a_blk, c), :] = (
                x_ref[ds(oa + na, c), :] + rv_a1[ds(na, c), :]
                + rv_a2[ds(0, c), :])
            a4p1[c] = exch(out_ref.at[ds(a_blk, c)],
                           out_ref.at[ds(a_blk, c)], y_nbr)
            a4p1[c].start()
            a3[c] = exch(out_ref.at[ds(a_blk, c)],
                         out_ref.at[ds(a_blk, c)], x_nbr)
            a3[c].start()

            b1[1, c].wait_recv()
            b2[c].wait_recv()
            out_ref[ds(b_blk, c), :] = (
                x_ref[ds(ob + nb, c), :] + rv_b1[ds(nb, c), :]
                + rv_b2[ds(0, c), :])
            b4p1[c] = exch(out_ref.at[ds(b_blk, c)],
                           out_ref.at[ds(b_blk, c)], x_nbr)
            b4p1[c].start()
            b3[c] = exch(out_ref.at[ds(b_blk, c)],
                         out_ref.at[ds(b_blk, c)], y_nbr)
            b3[c].start()

        a4p2, b4p2 = {}, {}
        for c in range(NC):
            a3[c].wait_recv()
            a4p2[c] = exch(out_ref.at[ds(oa + fa, c)],
                           out_ref.at[ds(oa + fa, c)], y_nbr)
            a4p2[c].start()
            b3[c].wait_recv()
            b4p2[c] = exch(out_ref.at[ds(ob + fb, c)],
                           out_ref.at[ds(ob + fb, c)], x_nbr)
            b4p2[c].start()

        for c in range(NC):
            a4p1[c].wait_recv()
            b4p1[c].wait_recv()
            a4p2[c].wait_recv()
            b4p2[c].wait_recv()

        for grp in (a1, b1, a2, b2, a3, b3, a4p1, b4p1, a4p2, b4p2):
            for r in grp.values():
                r.wait_send()

    n_sems = 12 * NC
    return pl.pallas_call(
        body,
        out_shape=jax.ShapeDtypeStruct((m, n), jnp.float32),
        in_specs=[pl.BlockSpec(memory_space=pltpu.VMEM)],
        out_specs=pl.BlockSpec(memory_space=pltpu.VMEM),
        scratch_shapes=[
            pltpu.VMEM((e, n), jnp.float32),
            pltpu.VMEM((e, n), jnp.float32),
            pltpu.VMEM((q, n), jnp.float32),
            pltpu.VMEM((q, n), jnp.float32),
            pltpu.VMEM((e, n), jnp.float32),
            pltpu.VMEM((e, n), jnp.float32),
            pltpu.SemaphoreType.DMA((n_sems,)),
            pltpu.SemaphoreType.DMA((n_sems,)),
        ],
        compiler_params=pltpu.CompilerParams(collective_id=0),
    )(x)
